# baseline (speedup 1.0000x reference)
import functools

import jax
import jax.numpy as jnp
from jax.experimental import pallas as pl
from jax.experimental.pallas import tpu as pltpu

B, S, H, E, K = 2, 2048, 2048, 64, 8
CAP = 1.25 * float(B * S) / float(E)
SINKHORN_ITERS = 10
M = B * S
M_BLK = 512


def _row_sum(v):
    acc = v[:, 0:128]
    for k in range(1, 16):
        acc = acc + v[:, 128 * k:128 * (k + 1)]
    w = acc[:, 0:8]
    for j in range(1, 16):
        w = w + acc[:, 8 * j:8 * j + 8]
    a = w[:, 0:4] + w[:, 4:8]
    b = a[:, 0:2] + a[:, 2:4]
    return b[:, 0:1] + b[:, 1:2]


def _mlp_kernel(x_ref, w1_ref, b1_ref, w2_ref, b2_ref, g_ref, be_ref, out_ref,
                xnb_ref):
    i = pl.program_id(0)
    nb = pl.num_programs(0) - 1

    @pl.when(i < nb)
    def _ln():
        x = x_ref[...]
        mean = _row_sum(x) * (1.0 / H)
        xc = x - mean
        var = _row_sum(xc * xc) * (1.0 / H)
        xn = xc / jnp.sqrt(var + 1e-5) * g_ref[...] + be_ref[...]
        xnb_ref[i % 2] = xn.astype(jnp.bfloat16)

    @pl.when(i > 0)
    def _mm():
        h = jnp.dot(xnb_ref[(i + 1) % 2], w1_ref[...],
                    preferred_element_type=jnp.float32) + b1_ref[...]
        h = jnp.maximum(h, 0.0)
        out_ref[...] = jnp.dot(h.astype(jnp.bfloat16), w2_ref[...],
                               preferred_element_type=jnp.float32) + b2_ref[...]


def _router_kernel(logits_ref, idx_ref, probs_ref, aux_ref):
    logits = logits_ref[...]
    m = jnp.max(logits, axis=-1, keepdims=True)
    assign = jnp.exp(logits - m)
    for _ in range(SINKHORN_ITERS):
        row = jnp.sum(assign, axis=-1, keepdims=True) + 1e-9
        assign = assign / row
        col = jnp.sum(assign, axis=1, keepdims=True) + 1e-9
        assign = assign / col * CAP
    row_final = jnp.sum(assign, axis=-1, keepdims=True)
    rp = assign / (row_final + 1e-9)
    iota = jax.lax.broadcasted_iota(jnp.int32, rp.shape, 2)
    vals, idxs = [], []
    cur = rp
    for _ in range(K):
        mx = jnp.max(cur, axis=-1, keepdims=True)
        sel = jnp.min(jnp.where(cur == mx, iota, E * 2), axis=-1, keepdims=True)
        vals.append(mx)
        idxs.append(sel)
        cur = jnp.where(iota == sel, -1.0, cur)
    tv = jnp.concatenate(vals, axis=-1)
    ti = jnp.concatenate(idxs, axis=-1)
    tv = tv / (jnp.sum(tv, axis=-1, keepdims=True) + 1e-9)
    idx_ref[...] = ti
    probs_ref[...] = tv
    col_final = jnp.sum(assign, axis=1)
    l_row = jnp.mean((row_final[..., 0] - 1.0) ** 2)
    l_col = jnp.mean((col_final - CAP) ** 2)
    aux_ref[...] = jnp.reshape(l_row + l_col, (1, 1))


@functools.partial(jax.jit, static_argnames=())
def kernel(x, W1, b1, W2, b2, gamma, beta):
    xf = x.reshape(M, H)
    nb = M // M_BLK
    logits = pl.pallas_call(
        _mlp_kernel,
        grid=(nb + 1,),
        in_specs=[
            pl.BlockSpec((M_BLK, H), lambda i: (jnp.minimum(i, nb - 1), 0)),
            pl.BlockSpec((H, H), lambda i: (0, 0)),
            pl.BlockSpec((1, H), lambda i: (0, 0)),
            pl.BlockSpec((H, E), lambda i: (0, 0)),
            pl.BlockSpec((1, E), lambda i: (0, 0)),
            pl.BlockSpec((1, H), lambda i: (0, 0)),
            pl.BlockSpec((1, H), lambda i: (0, 0)),
        ],
        out_specs=pl.BlockSpec((M_BLK, E), lambda i: (jnp.maximum(i - 1, 0), 0)),
        out_shape=jax.ShapeDtypeStruct((M, E), jnp.float32),
        scratch_shapes=[pltpu.VMEM((2, M_BLK, H), jnp.bfloat16)],
    )(xf, W1.astype(jnp.bfloat16), b1.reshape(1, H),
      W2.astype(jnp.bfloat16), b2.reshape(1, E),
      gamma.reshape(1, H), beta.reshape(1, H))

    ti, tv, aux = pl.pallas_call(
        _router_kernel,
        out_shape=(
            jax.ShapeDtypeStruct((B, S, K), jnp.int32),
            jax.ShapeDtypeStruct((B, S, K), jnp.float32),
            jax.ShapeDtypeStruct((1, 1), jnp.float32),
        ),
        out_specs=(
            pl.BlockSpec((B, S, K), lambda: (0, 0, 0)),
            pl.BlockSpec((B, S, K), lambda: (0, 0, 0)),
            pl.BlockSpec((1, 1), lambda: (0, 0)),
        ),
        in_specs=[pl.BlockSpec((B, S, E), lambda: (0, 0, 0))],
    )(logits.reshape(B, S, E))
    return ti, tv, aux[0, 0]


# speedup vs baseline: 1.1175x; 1.1175x over previous
import functools

import jax
import jax.numpy as jnp
from jax.experimental import pallas as pl
from jax.experimental.pallas import tpu as pltpu

B, S, H, E, K = 2, 2048, 2048, 64, 8
CAP = 1.25 * float(B * S) / float(E)
SINKHORN_ITERS = 10
M = B * S
M_BLK = 512


def _row_sum(v):
    acc = v[:, 0:128]
    for k in range(1, 16):
        acc = acc + v[:, 128 * k:128 * (k + 1)]
    w = acc[:, 0:8]
    for j in range(1, 16):
        w = w + acc[:, 8 * j:8 * j + 8]
    a = w[:, 0:4] + w[:, 4:8]
    b = a[:, 0:2] + a[:, 2:4]
    return b[:, 0:1] + b[:, 1:2]


def _mlp_kernel(x_ref, w1_ref, b1_ref, w2_ref, b2_ref, g_ref, be_ref, out_ref,
                xnb_ref):
    i = pl.program_id(0)
    x = x_ref[...]
    mean = _row_sum(x) * (1.0 / H)
    xc = x - mean
    var = _row_sum(xc * xc) * (1.0 / H)
    xn = xc / jnp.sqrt(var + 1e-5) * g_ref[...] + be_ref[...]
    h = jnp.dot(xnb_ref[(i + 1) % 2], w1_ref[...],
                preferred_element_type=jnp.float32) + b1_ref[...]
    h = jnp.maximum(h, 0.0)
    out_ref[...] = jnp.dot(h.astype(jnp.bfloat16), w2_ref[...],
                           preferred_element_type=jnp.float32) + b2_ref[...]
    xnb_ref[i % 2] = xn.astype(jnp.bfloat16)


def _router_kernel(logits_ref, idx_ref, probs_ref, aux_ref):
    logits = logits_ref[...]
    m = jnp.max(logits, axis=-1, keepdims=True)
    assign = jnp.exp(logits - m)
    for _ in range(SINKHORN_ITERS):
        row = jnp.sum(assign, axis=-1, keepdims=True) + 1e-9
        assign = assign / row
        col = jnp.sum(assign, axis=1, keepdims=True) + 1e-9
        assign = assign / col * CAP
    row_final = jnp.sum(assign, axis=-1, keepdims=True)
    rp = assign / (row_final + 1e-9)
    iota = jax.lax.broadcasted_iota(jnp.int32, rp.shape, 2)
    vals, idxs = [], []
    cur = rp
    for _ in range(K):
        mx = jnp.max(cur, axis=-1, keepdims=True)
        sel = jnp.min(jnp.where(cur == mx, iota, E * 2), axis=-1, keepdims=True)
        vals.append(mx)
        idxs.append(sel)
        cur = jnp.where(iota == sel, -1.0, cur)
    tv = jnp.concatenate(vals, axis=-1)
    ti = jnp.concatenate(idxs, axis=-1)
    tv = tv / (jnp.sum(tv, axis=-1, keepdims=True) + 1e-9)
    idx_ref[...] = ti
    probs_ref[...] = tv
    col_final = jnp.sum(assign, axis=1)
    l_row = jnp.mean((row_final[..., 0] - 1.0) ** 2)
    l_col = jnp.mean((col_final - CAP) ** 2)
    aux_ref[...] = jnp.reshape(l_row + l_col, (1, 1))


@functools.partial(jax.jit, static_argnames=())
def kernel(x, W1, b1, W2, b2, gamma, beta):
    xf = x.reshape(M, H)
    nb = M // M_BLK
    logits = pl.pallas_call(
        _mlp_kernel,
        grid=(nb + 1,),
        in_specs=[
            pl.BlockSpec((M_BLK, H), lambda i: (jnp.minimum(i, nb - 1), 0)),
            pl.BlockSpec((H, H), lambda i: (0, 0)),
            pl.BlockSpec((1, H), lambda i: (0, 0)),
            pl.BlockSpec((H, E), lambda i: (0, 0)),
            pl.BlockSpec((1, E), lambda i: (0, 0)),
            pl.BlockSpec((1, H), lambda i: (0, 0)),
            pl.BlockSpec((1, H), lambda i: (0, 0)),
        ],
        out_specs=pl.BlockSpec((M_BLK, E), lambda i: (jnp.maximum(i - 1, 0), 0)),
        out_shape=jax.ShapeDtypeStruct((M, E), jnp.float32),
        scratch_shapes=[pltpu.VMEM((2, M_BLK, H), jnp.bfloat16)],
    )(xf, W1.astype(jnp.bfloat16), b1.reshape(1, H),
      W2.astype(jnp.bfloat16), b2.reshape(1, E),
      gamma.reshape(1, H), beta.reshape(1, H))

    ti, tv, aux = pl.pallas_call(
        _router_kernel,
        out_shape=(
            jax.ShapeDtypeStruct((B, S, K), jnp.int32),
            jax.ShapeDtypeStruct((B, S, K), jnp.float32),
            jax.ShapeDtypeStruct((1, 1), jnp.float32),
        ),
        out_specs=(
            pl.BlockSpec((B, S, K), lambda: (0, 0, 0)),
            pl.BlockSpec((B, S, K), lambda: (0, 0, 0)),
            pl.BlockSpec((1, 1), lambda: (0, 0)),
        ),
        in_specs=[pl.BlockSpec((B, S, E), lambda: (0, 0, 0))],
    )(logits.reshape(B, S, E))
    return ti, tv, aux[0, 0]


# speedup vs baseline: 1.1416x; 1.0216x over previous
import functools

import jax
import jax.numpy as jnp
from jax.experimental import pallas as pl
from jax.experimental.pallas import tpu as pltpu

B, S, H, E, K = 2, 2048, 2048, 64, 8
CAP = 1.25 * float(B * S) / float(E)
SINKHORN_ITERS = 10
M = B * S
M_BLK = 512


def _row_sum(v):
    acc = v[:, 0:128]
    for k in range(1, 16):
        acc = acc + v[:, 128 * k:128 * (k + 1)]
    w = acc[:, 0:8]
    for j in range(1, 16):
        w = w + acc[:, 8 * j:8 * j + 8]
    a = w[:, 0:4] + w[:, 4:8]
    b = a[:, 0:2] + a[:, 2:4]
    return b[:, 0:1] + b[:, 1:2]


def _fused_kernel(x_ref, w1_ref, b1_ref, w2_ref, b2_ref, g_ref, be_ref,
                  idx_ref, probs_ref, aux_ref,
                  xnb_ref, logits_ref):
    i = pl.program_id(0)
    nb = pl.num_programs(0) - 1
    x = x_ref[...]
    mean = _row_sum(x) * (1.0 / H)
    xc = x - mean
    var = _row_sum(xc * xc) * (1.0 / H)
    xn = xc / jnp.sqrt(var + 1e-5) * g_ref[...] + be_ref[...]
    h = jnp.dot(xnb_ref[(i + 1) % 2], w1_ref[...],
                preferred_element_type=jnp.float32) + b1_ref[...]
    h = jnp.maximum(h, 0.0)
    lg = jnp.dot(h.astype(jnp.bfloat16), w2_ref[...],
                 preferred_element_type=jnp.float32) + b2_ref[...]
    prev = jnp.maximum(i - 1, 0)
    logits_ref[pl.ds(prev * M_BLK, M_BLK), :] = lg
    xnb_ref[i % 2] = xn.astype(jnp.bfloat16)

    @pl.when(i == nb)
    def _router():
        logits = logits_ref[...].reshape(B, S, E)
        m = jnp.max(logits, axis=-1, keepdims=True)
        assign = jnp.exp(logits - m)
        for _ in range(SINKHORN_ITERS):
            row = jnp.sum(assign, axis=-1, keepdims=True) + 1e-9
            assign = assign / row
            col = jnp.sum(assign, axis=1, keepdims=True) + 1e-9
            assign = assign / col * CAP
        row_final = jnp.sum(assign, axis=-1, keepdims=True)
        rp = assign / (row_final + 1e-9)
        iota = jax.lax.broadcasted_iota(jnp.int32, rp.shape, 2)
        vals, idxs = [], []
        cur = rp
        for _ in range(K):
            mx = jnp.max(cur, axis=-1, keepdims=True)
            sel = jnp.min(jnp.where(cur == mx, iota, E * 2), axis=-1,
                          keepdims=True)
            vals.append(mx)
            idxs.append(sel)
            cur = jnp.where(iota == sel, -1.0, cur)
        tv = jnp.concatenate(vals, axis=-1)
        ti = jnp.concatenate(idxs, axis=-1)
        tv = tv / (jnp.sum(tv, axis=-1, keepdims=True) + 1e-9)
        idx_ref[...] = ti
        probs_ref[...] = tv
        col_final = jnp.sum(assign, axis=1)
        l_row = jnp.mean((row_final[..., 0] - 1.0) ** 2)
        l_col = jnp.mean((col_final - CAP) ** 2)
        aux_ref[...] = jnp.reshape(l_row + l_col, (1, 1))


@functools.partial(jax.jit, static_argnames=())
def kernel(x, W1, b1, W2, b2, gamma, beta):
    xf = x.reshape(M, H)
    nb = M // M_BLK
    ti, tv, aux = pl.pallas_call(
        _fused_kernel,
        grid=(nb + 1,),
        in_specs=[
            pl.BlockSpec((M_BLK, H), lambda i: (jnp.minimum(i, nb - 1), 0)),
            pl.BlockSpec((H, H), lambda i: (0, 0)),
            pl.BlockSpec((1, H), lambda i: (0, 0)),
            pl.BlockSpec((H, E), lambda i: (0, 0)),
            pl.BlockSpec((1, E), lambda i: (0, 0)),
            pl.BlockSpec((1, H), lambda i: (0, 0)),
            pl.BlockSpec((1, H), lambda i: (0, 0)),
        ],
        out_specs=(
            pl.BlockSpec((B, S, K), lambda i: (0, 0, 0)),
            pl.BlockSpec((B, S, K), lambda i: (0, 0, 0)),
            pl.BlockSpec((1, 1), lambda i: (0, 0)),
        ),
        out_shape=(
            jax.ShapeDtypeStruct((B, S, K), jnp.int32),
            jax.ShapeDtypeStruct((B, S, K), jnp.float32),
            jax.ShapeDtypeStruct((1, 1), jnp.float32),
        ),
        scratch_shapes=[
            pltpu.VMEM((2, M_BLK, H), jnp.bfloat16),
            pltpu.VMEM((M, E), jnp.float32),
        ],
    )(xf, W1.astype(jnp.bfloat16), b1.reshape(1, H),
      W2.astype(jnp.bfloat16), b2.reshape(1, E),
      gamma.reshape(1, H), beta.reshape(1, H))
    return ti, tv, aux[0, 0]
